# baseline (speedup 1.0000x reference)
import functools

import jax
import jax.numpy as jnp
from jax import lax
from jax.experimental import pallas as pl
from jax.experimental.pallas import tpu as pltpu
from jax.experimental.pallas import tpu_sc as plsc

N_TABLE = 100000
D = 128
N_CLASSES = 47
N1, N2, N3 = 12000, 4000, 1024
E0, E1, E2 = 192000, 64000, 16384

NC, NS = 2, 16
NW = NC * NS
CH = 128

E0P = 47 * NW * CH
E1P = 16 * NW * CH
E2P = E2
H0_ROWS = 12288
OUT0, OUT1, OUT2 = 4096, 1024, 1024
ACC0, ACC1, ACC2 = OUT0 + 128, OUT1 + 128, OUT2
TRASH0, TRASH1 = OUT0, OUT1


def _cdiv(a, b):
    return (a + b - 1) // b


def _make_gather(n_rows):
    per_tile = n_rows // NW
    n_chunks = per_tile // CH

    mesh = plsc.VectorSubcoreMesh(
        core_axis_name="c", subcore_axis_name="s",
        num_cores=NC, num_subcores=NS)

    def body(table_hbm, idx_hbm, out_hbm, idx_v, rows_v, sem):
        wid = lax.axis_index("s") * NC + lax.axis_index("c")
        base = wid * per_tile

        def chunk_body(t, c):
            b = base + t * CH
            pltpu.sync_copy(idx_hbm.at[pl.ds(b, CH)], idx_v)
            pltpu.async_copy(table_hbm.at[idx_v], rows_v, sem).wait()
            pltpu.sync_copy(rows_v, out_hbm.at[pl.ds(b, CH), :])
            return c
        lax.fori_loop(0, n_chunks, chunk_body, 0)

    return pl.kernel(
        body,
        out_type=jax.ShapeDtypeStruct((n_rows, D), jnp.float32),
        mesh=mesh,
        scratch_types=[
            pltpu.VMEM((CH,), jnp.int32),
            pltpu.VMEM((CH, D), jnp.float32),
            pltpu.SemaphoreType.DMA,
        ])


def _make_seg(e_pad, n_acc, n_out):
    per_tile = e_pad // NW
    n_chunks = per_tile // CH
    n_zslices = n_acc // 128
    nzc = _cdiv(n_zslices, NS)
    out_per_tile = n_out // NS

    mesh = plsc.VectorSubcoreMesh(
        core_axis_name="c", subcore_axis_name="s",
        num_cores=NC, num_subcores=NS)

    out_type = [
        jax.ShapeDtypeStruct((NC, n_out, 128), jnp.float32),
        jax.ShapeDtypeStruct((NC, n_out, 128), jnp.float32),
    ]
    scratch = [
        pltpu.VMEM((CH,), jnp.int32),
        pltpu.VMEM((CH,), jnp.int32),
        pltpu.VMEM((CH, 128), jnp.float32),
        pltpu.VMEM((CH, 128), jnp.float32),
        pltpu.VMEM((128, 128), jnp.float32),
        pltpu.VMEM_SHARED((n_acc, 128), jnp.float32),
        pltpu.VMEM_SHARED((n_acc, 128), jnp.float32),
        pltpu.SemaphoreType.DMA,
    ]

    def body(h_hbm, src_hbm, dst_hbm, psum_o, pdeg_o,
             src_v, dst_v, rows_v, ones_v, zbuf,
             acc_sh, deg_sh, sem):
        cid = lax.axis_index("c")
        sid = lax.axis_index("s")
        wid = sid * NC + cid

        zero16 = jnp.zeros((16,), jnp.float32)
        lane0 = jnp.where(lax.iota(jnp.int32, 16) == 0, 1.0, 0.0)

        def init_body(i, c):
            for j in range(8):
                zbuf[i, pl.ds(j * 16, 16)] = zero16
                ones_v[i, pl.ds(j * 16, 16)] = lane0 if j == 0 else zero16
            return c
        lax.fori_loop(0, 128, init_body, 0)

        def zbody(t, c):
            sidx = sid * nzc + t

            @pl.when(sidx < n_zslices)
            def _():
                pltpu.sync_copy(zbuf, acc_sh.at[pl.ds(sidx * 128, 128), :])
                pltpu.sync_copy(zbuf, deg_sh.at[pl.ds(sidx * 128, 128), :])
            return c
        lax.fori_loop(0, nzc, zbody, 0)
        plsc.subcore_barrier()

        ebase = wid * per_tile

        def chunk_body(t, c):
            base = ebase + t * CH
            pltpu.sync_copy(src_hbm.at[pl.ds(base, CH)], src_v)
            pltpu.sync_copy(dst_hbm.at[pl.ds(base, CH)], dst_v)
            pltpu.async_copy(h_hbm.at[src_v], rows_v, sem).wait()
            pltpu.sync_copy(rows_v, acc_sh.at[dst_v], add=True)
            pltpu.sync_copy(ones_v, deg_sh.at[dst_v], add=True)
            return c
        lax.fori_loop(0, n_chunks, chunk_body, 0)
        plsc.subcore_barrier()

        obase = sid * out_per_tile
        pltpu.sync_copy(acc_sh.at[pl.ds(obase, out_per_tile), :],
                        psum_o.at[cid, pl.ds(obase, out_per_tile), :])
        pltpu.sync_copy(deg_sh.at[pl.ds(obase, out_per_tile), :],
                        pdeg_o.at[cid, pl.ds(obase, out_per_tile), :])

    return pl.kernel(body, out_type=tuple(out_type), mesh=mesh,
                     scratch_types=scratch)


_GATHER_H0 = _make_gather(H0_ROWS)
_SEG0 = _make_seg(E0P, ACC0, OUT0)
_SEG1 = _make_seg(E1P, ACC1, OUT1)
_SEG2 = _make_seg(E2P, ACC2, OUT2)


def _tc_body(hs_ref, ps_ref, pd_ref, ws_ref, wn_ref, b_ref, o_ref, *, relu):
    s = ps_ref[0] + ps_ref[1]
    d = pd_ref[0, :, 0:1] + pd_ref[1, :, 0:1]
    agg = s * (1.0 / jnp.maximum(d, 1.0))
    acc = jnp.dot(hs_ref[...], ws_ref[...], preferred_element_type=jnp.float32)
    acc = acc + jnp.dot(agg, wn_ref[...], preferred_element_type=jnp.float32)
    acc = acc + b_ref[...]
    if relu:
        acc = jnp.maximum(acc, 0.0)
    o_ref[...] = acc


def _tc_layer(hself, psum, pdeg, w_self, w_neigh, b2d, relu, m_out, bm=512):
    return pl.pallas_call(
        functools.partial(_tc_body, relu=relu),
        grid=(m_out // bm,),
        in_specs=[
            pl.BlockSpec((bm, 128), lambda i: (i, 0)),
            pl.BlockSpec((2, bm, 128), lambda i: (0, i, 0)),
            pl.BlockSpec((2, bm, 128), lambda i: (0, i, 0)),
            pl.BlockSpec((128, 128), lambda i: (0, 0)),
            pl.BlockSpec((128, 128), lambda i: (0, 0)),
            pl.BlockSpec((1, 128), lambda i: (0, 0)),
        ],
        out_specs=pl.BlockSpec((bm, 128), lambda i: (i, 0)),
        out_shape=jax.ShapeDtypeStruct((m_out, 128), jnp.float32),
    )(hself, psum, pdeg, w_self, w_neigh, b2d)


@jax.jit
def kernel(input_nodes, edge_index_0, edge_index_1, edge_index_2, emb_table,
           W_neigh_0, W_self_0, b_0, W_neigh_1, W_self_1, b_1,
           W_neigh_2, W_self_2, b_2):
    d0 = edge_index_0[1]
    live0 = d0 < OUT0
    src0 = jnp.concatenate(
        [jnp.where(live0, edge_index_0[0], 0),
         jnp.zeros((E0P - E0,), jnp.int32)])
    dst0 = jnp.concatenate(
        [jnp.where(live0, d0, TRASH0),
         jnp.full((E0P - E0,), TRASH0, jnp.int32)])
    d1 = edge_index_1[1]
    live1 = d1 < OUT1
    src1 = jnp.concatenate(
        [jnp.where(live1, edge_index_1[0], 0),
         jnp.zeros((E1P - E1,), jnp.int32)])
    dst1 = jnp.concatenate(
        [jnp.where(live1, d1, TRASH1),
         jnp.full((E1P - E1,), TRASH1, jnp.int32)])

    h0 = _GATHER_H0(emb_table, input_nodes)

    psum0, pdeg0 = _SEG0(h0, src0, dst0)
    h1 = _tc_layer(h0, psum0, pdeg0, W_self_0, W_neigh_0,
                   b_0.reshape(1, 128), True, OUT0)

    psum1, pdeg1 = _SEG1(h1, src1, dst1)
    h2 = _tc_layer(h1, psum1, pdeg1, W_self_1, W_neigh_1,
                   b_1.reshape(1, 128), True, OUT1)

    psum2, pdeg2 = _SEG2(h2, edge_index_2[0], edge_index_2[1])
    ws2 = jnp.pad(W_self_2, ((0, 0), (0, 128 - N_CLASSES)))
    wn2 = jnp.pad(W_neigh_2, ((0, 0), (0, 128 - N_CLASSES)))
    b2 = jnp.pad(b_2, (0, 128 - N_CLASSES)).reshape(1, 128)
    h3 = _tc_layer(h2, psum2, pdeg2, ws2, wn2, b2, False, OUT2, bm=512)

    return h3[:, :N_CLASSES]


# speedup vs baseline: 1.0046x; 1.0046x over previous
import functools

import jax
import jax.numpy as jnp
from jax import lax
from jax.experimental import pallas as pl
from jax.experimental.pallas import tpu as pltpu
from jax.experimental.pallas import tpu_sc as plsc

N_TABLE = 100000
D = 128
N_CLASSES = 47
N1, N2, N3 = 12000, 4000, 1024
E0, E1, E2 = 192000, 64000, 16384

NC, NS = 2, 16
NW = NC * NS
CH = 128

E0P = 47 * NW * CH
E1P = 16 * NW * CH
E2P = E2
H0_ROWS = 12288
OUT0, OUT1, OUT2 = 4096, 1024, 1024
ACC0, ACC1, ACC2 = OUT0 + 128, OUT1 + 128, OUT2
TRASH0, TRASH1 = OUT0, OUT1


def _cdiv(a, b):
    return (a + b - 1) // b


def _make_gather(n_rows):
    per_tile = n_rows // NW
    n_chunks = per_tile // CH

    mesh = plsc.VectorSubcoreMesh(
        core_axis_name="c", subcore_axis_name="s",
        num_cores=NC, num_subcores=NS)

    def body(table_hbm, idx_hbm, out_hbm, idx_v, rows_v, sem):
        wid = lax.axis_index("s") * NC + lax.axis_index("c")
        base = wid * per_tile

        def chunk_body(t, c):
            b = base + t * CH
            pltpu.sync_copy(idx_hbm.at[pl.ds(b, CH)], idx_v)
            pltpu.async_copy(table_hbm.at[idx_v], rows_v, sem).wait()
            pltpu.sync_copy(rows_v, out_hbm.at[pl.ds(b, CH), :])
            return c
        lax.fori_loop(0, n_chunks, chunk_body, 0)

    return pl.kernel(
        body,
        out_type=jax.ShapeDtypeStruct((n_rows, D), jnp.float32),
        mesh=mesh,
        scratch_types=[
            pltpu.VMEM((CH,), jnp.int32),
            pltpu.VMEM((CH, D), jnp.float32),
            pltpu.SemaphoreType.DMA,
        ])


def _make_seg(e_pad, n_acc, n_out):
    per_tile = e_pad // NW
    n_chunks = per_tile // CH
    n_zslices = n_acc // 128
    nzc = _cdiv(n_zslices, NS)
    out_per_tile = n_out // NS

    mesh = plsc.VectorSubcoreMesh(
        core_axis_name="c", subcore_axis_name="s",
        num_cores=NC, num_subcores=NS)

    out_type = [
        jax.ShapeDtypeStruct((NC, n_out, 128), jnp.float32),
        jax.ShapeDtypeStruct((NC, n_out, 128), jnp.float32),
    ]
    scratch = [
        pltpu.VMEM((CH,), jnp.int32),
        pltpu.VMEM((CH,), jnp.int32),
        pltpu.VMEM((CH, 128), jnp.float32),
        pltpu.VMEM((CH, 128), jnp.float32),
        pltpu.VMEM((128, 128), jnp.float32),
        pltpu.VMEM_SHARED((n_acc, 128), jnp.float32),
        pltpu.VMEM_SHARED((n_acc, 128), jnp.float32),
        pltpu.SemaphoreType.DMA,
    ]

    def body(h_hbm, src_hbm, dst_hbm, psum_o, pdeg_o,
             src_v, dst_v, rows_v, ones_v, zbuf,
             acc_sh, deg_sh, sem):
        cid = lax.axis_index("c")
        sid = lax.axis_index("s")
        wid = sid * NC + cid

        zero16 = jnp.zeros((16,), jnp.float32)
        lane0 = jnp.where(lax.iota(jnp.int32, 16) == 0, 1.0, 0.0)

        def init_body(i, c):
            for j in range(8):
                zbuf[i, pl.ds(j * 16, 16)] = zero16
                ones_v[i, pl.ds(j * 16, 16)] = lane0 if j == 0 else zero16
            return c
        lax.fori_loop(0, 128, init_body, 0)

        def zbody(t, c):
            sidx = sid * nzc + t

            @pl.when(sidx < n_zslices)
            def _():
                pltpu.sync_copy(zbuf, acc_sh.at[pl.ds(sidx * 128, 128), :])
                pltpu.sync_copy(zbuf, deg_sh.at[pl.ds(sidx * 128, 128), :])
            return c
        lax.fori_loop(0, nzc, zbody, 0)
        plsc.subcore_barrier()

        ebase = wid * per_tile

        def chunk_body(t, c):
            base = ebase + t * CH
            pltpu.sync_copy(src_hbm.at[pl.ds(base, CH)], src_v)
            pltpu.sync_copy(dst_hbm.at[pl.ds(base, CH)], dst_v)
            pltpu.async_copy(h_hbm.at[src_v], rows_v, sem).wait()
            pltpu.sync_copy(rows_v, acc_sh.at[dst_v], add=True)
            pltpu.sync_copy(ones_v, deg_sh.at[dst_v], add=True)
            return c
        lax.fori_loop(0, n_chunks, chunk_body, 0)
        plsc.subcore_barrier()

        obase = sid * out_per_tile
        pltpu.sync_copy(acc_sh.at[pl.ds(obase, out_per_tile), :],
                        psum_o.at[cid, pl.ds(obase, out_per_tile), :])
        pltpu.sync_copy(deg_sh.at[pl.ds(obase, out_per_tile), :],
                        pdeg_o.at[cid, pl.ds(obase, out_per_tile), :])

    return pl.kernel(body, out_type=tuple(out_type), mesh=mesh,
                     scratch_types=scratch)


_GATHER_H0 = _make_gather(H0_ROWS)
_SEG0 = _make_seg(E0P, ACC0, OUT0)
_SEG1 = _make_seg(E1P, ACC1, OUT1)
_SEG2 = _make_seg(E2P, ACC2, OUT2)


def _tc_body(hs_ref, ps_ref, pd_ref, ws_ref, wn_ref, b_ref, o_ref, *, relu):
    s = ps_ref[0] + ps_ref[1]
    d = pd_ref[0, :, 0:1] + pd_ref[1, :, 0:1]
    agg = s * (1.0 / jnp.maximum(d, 1.0))
    acc = jnp.dot(hs_ref[...], ws_ref[...], preferred_element_type=jnp.float32)
    acc = acc + jnp.dot(agg, wn_ref[...], preferred_element_type=jnp.float32)
    acc = acc + b_ref[...]
    if relu:
        acc = jnp.maximum(acc, 0.0)
    o_ref[...] = acc


def _tc_layer(hself, psum, pdeg, w_self, w_neigh, b2d, relu, m_out, bm=512):
    return pl.pallas_call(
        functools.partial(_tc_body, relu=relu),
        grid=(m_out // bm,),
        in_specs=[
            pl.BlockSpec((bm, 128), lambda i: (i, 0)),
            pl.BlockSpec((2, bm, 128), lambda i: (0, i, 0)),
            pl.BlockSpec((2, bm, 128), lambda i: (0, i, 0)),
            pl.BlockSpec((128, 128), lambda i: (0, 0)),
            pl.BlockSpec((128, 128), lambda i: (0, 0)),
            pl.BlockSpec((1, 128), lambda i: (0, 0)),
        ],
        out_specs=pl.BlockSpec((bm, 128), lambda i: (i, 0)),
        out_shape=jax.ShapeDtypeStruct((m_out, 128), jnp.float32),
    )(hself, psum, pdeg, w_self, w_neigh, b2d)


@jax.jit
def kernel(input_nodes, edge_index_0, edge_index_1, edge_index_2, emb_table,
           W_neigh_0, W_self_0, b_0, W_neigh_1, W_self_1, b_1,
           W_neigh_2, W_self_2, b_2):
    spread0 = TRASH0 + (jnp.arange(E0, dtype=jnp.int32) & 127)
    spread0p = TRASH0 + (jnp.arange(E0P - E0, dtype=jnp.int32) & 127)
    d0 = edge_index_0[1]
    live0 = d0 < OUT0
    src0 = jnp.concatenate(
        [jnp.where(live0, edge_index_0[0], 0),
         jnp.zeros((E0P - E0,), jnp.int32)])
    dst0 = jnp.concatenate([jnp.where(live0, d0, spread0), spread0p])
    spread1 = TRASH1 + (jnp.arange(E1, dtype=jnp.int32) & 127)
    spread1p = TRASH1 + (jnp.arange(E1P - E1, dtype=jnp.int32) & 127)
    d1 = edge_index_1[1]
    live1 = d1 < OUT1
    src1 = jnp.concatenate(
        [jnp.where(live1, edge_index_1[0], 0),
         jnp.zeros((E1P - E1,), jnp.int32)])
    dst1 = jnp.concatenate([jnp.where(live1, d1, spread1), spread1p])

    h0 = _GATHER_H0(emb_table, input_nodes)

    psum0, pdeg0 = _SEG0(h0, src0, dst0)
    h1 = _tc_layer(h0, psum0, pdeg0, W_self_0, W_neigh_0,
                   b_0.reshape(1, 128), True, OUT0)

    psum1, pdeg1 = _SEG1(h1, src1, dst1)
    h2 = _tc_layer(h1, psum1, pdeg1, W_self_1, W_neigh_1,
                   b_1.reshape(1, 128), True, OUT1)

    psum2, pdeg2 = _SEG2(h2, edge_index_2[0], edge_index_2[1])
    ws2 = jnp.pad(W_self_2, ((0, 0), (0, 128 - N_CLASSES)))
    wn2 = jnp.pad(W_neigh_2, ((0, 0), (0, 128 - N_CLASSES)))
    b2 = jnp.pad(b_2, (0, 128 - N_CLASSES)).reshape(1, 128)
    h3 = _tc_layer(h2, psum2, pdeg2, ws2, wn2, b2, False, OUT2, bm=512)

    return h3[:, :N_CLASSES]
